# baseline (device time: 122008 ns/iter reference)
import jax
import jax.numpy as jnp
from jax import lax
from jax.experimental import pallas as pl
from jax.experimental.pallas import tpu as pltpu

N_DEV = 32
LOG2 = 5
B, Sq, Hq, Hkv, Dh = 2, 256, 8, 2, 64
Dm, Dq = 768, 512
BH = B * Hq
ROWS = 72
GQA = Hq // Hkv


def kernel(x, Wq, Wo, K_ext, V_ext):
    def body(x_ref, wq_ref, wo_ref, k_ref, v_ref, out_ref,
             sbuf, rbuf, send_sems, recv_sems):
        my = lax.axis_index("i")

        barrier = pltpu.get_barrier_semaphore()
        for k in range(LOG2):
            pl.semaphore_signal(
                barrier, inc=1,
                device_id=(my ^ (1 << k),),
                device_id_type=pl.DeviceIdType.MESH,
            )
        pl.semaphore_wait(barrier, LOG2)

        for b in range(B):
            qb = jnp.dot(x_ref[b], wq_ref[...],
                         preferred_element_type=jnp.float32)
            for h in range(Hq):
                kh = h // GQA
                q = qb[:, h * Dh:(h + 1) * Dh]
                kk = k_ref[b, :, kh, :]
                vv = v_ref[b, :, kh, :]
                sT = lax.dot_general(
                    kk, q, (((1,), (1,)), ((), ())),
                    preferred_element_type=jnp.float32) * 0.125
                m = jnp.max(sT, axis=0, keepdims=True)
                p = jnp.exp(sT - m)
                l = jnp.sum(p, axis=0, keepdims=True)
                oT = lax.dot_general(
                    vv, p, (((0,), (0,)), ((), ())),
                    preferred_element_type=jnp.float32)
                i = b * Hq + h
                sbuf[0, i, 0:Dh, :] = oT
                sbuf[0, i, Dh:Dh + 1, :] = m
                sbuf[0, i, Dh + 1:Dh + 2, :] = l
                sbuf[0, i, Dh + 2:ROWS, :] = jnp.zeros((ROWS - Dh - 2, Sq),
                                                       jnp.float32)

        for k in range(LOG2):
            s = k % 2
            d = (k + 1) % 2
            rdma = pltpu.make_async_remote_copy(
                src_ref=sbuf.at[s],
                dst_ref=rbuf.at[k],
                send_sem=send_sems.at[k],
                recv_sem=recv_sems.at[k],
                device_id=(my ^ (1 << k),),
                device_id_type=pl.DeviceIdType.MESH,
            )
            rdma.start()
            rdma.wait()

            A = sbuf[s]
            R = rbuf[k]
            a_m = A[:, Dh:Dh + 1, :]
            r_m = R[:, Dh:Dh + 1, :]
            a_l = A[:, Dh + 1:Dh + 2, :]
            r_l = R[:, Dh + 1:Dh + 2, :]
            m_new = jnp.maximum(a_m, r_m)
            aa = jnp.exp(a_m - m_new)
            ar = jnp.exp(r_m - m_new)
            sbuf[d, :, 0:Dh, :] = A[:, 0:Dh, :] * aa + R[:, 0:Dh, :] * ar
            sbuf[d, :, Dh:Dh + 1, :] = m_new
            sbuf[d, :, Dh + 1:Dh + 2, :] = a_l * aa + r_l * ar

        f = LOG2 % 2
        for b in range(B):
            acc = jnp.zeros((Sq, Dm), dtype=jnp.float32)
            for h in range(Hq):
                i = b * Hq + h
                on = sbuf[f, i, 0:Dh, :] / sbuf[f, i, Dh + 1:Dh + 2, :]
                acc = acc + lax.dot_general(
                    on, wo_ref[h * Dh:(h + 1) * Dh, :],
                    (((0,), (0,)), ((), ())),
                    preferred_element_type=jnp.float32)
            out_ref[b, :, :] = acc

    return pl.pallas_call(
        body,
        out_shape=jax.ShapeDtypeStruct((B, Sq, Dm), jnp.float32),
        in_specs=[pl.BlockSpec(memory_space=pltpu.VMEM)] * 5,
        out_specs=pl.BlockSpec(memory_space=pltpu.VMEM),
        scratch_shapes=[
            pltpu.VMEM((2, BH, ROWS, Sq), jnp.float32),
            pltpu.VMEM((LOG2, BH, ROWS, Sq), jnp.float32),
            pltpu.SemaphoreType.DMA((LOG2,)),
            pltpu.SemaphoreType.DMA((LOG2,)),
        ],
        compiler_params=pltpu.CompilerParams(collective_id=0),
    )(x, Wq, Wo, K_ext, V_ext)


# device time: 66878 ns/iter; 1.8243x vs baseline; 1.8243x over previous
import jax
import jax.numpy as jnp
from jax import lax
from jax.experimental import pallas as pl
from jax.experimental.pallas import tpu as pltpu

N_DEV = 32
LOG2 = 5
B, Sq, Hq, Hkv, Dh = 2, 256, 8, 2, 64
Dm, Dq = 768, 512
BH = B * Hq
ROWS = 72
HLF = Sq // 2
GQA = Hq // Hkv


def kernel(x, Wq, Wo, K_ext, V_ext):
    def body(x_ref, wq_ref, wo_ref, k_ref, v_ref, out_ref,
             sbuf, rbuf, send_sems, recv_sems):
        my = lax.axis_index("i")

        barrier = pltpu.get_barrier_semaphore()
        for k in range(LOG2):
            pl.semaphore_signal(
                barrier, inc=1,
                device_id=(my ^ (1 << k),),
                device_id_type=pl.DeviceIdType.MESH,
            )
        pl.semaphore_wait(barrier, LOG2)

        for b in range(B):
            qb = jnp.dot(x_ref[b], wq_ref[...],
                         preferred_element_type=jnp.float32)
            for h in range(Hq):
                kh = h // GQA
                q = qb[:, h * Dh:(h + 1) * Dh]
                kk = k_ref[b, :, kh, :]
                vv = v_ref[b, :, kh, :]
                sT = lax.dot_general(
                    kk, q, (((1,), (1,)), ((), ())),
                    preferred_element_type=jnp.float32) * 0.125
                m = jnp.max(sT, axis=0, keepdims=True)
                p = jnp.exp(sT - m)
                l = jnp.sum(p, axis=0, keepdims=True)
                oT = lax.dot_general(
                    vv, p, (((0,), (0,)), ((), ())),
                    preferred_element_type=jnp.float32)
                i = b * Hq + h
                for j in range(2):
                    pc = 2 * i + j
                    cols = slice(j * HLF, (j + 1) * HLF)
                    sbuf[pc, 0:Dh, :] = oT[:, cols]
                    sbuf[pc, Dh:Dh + 1, :] = m[:, cols]
                    sbuf[pc, Dh + 1:Dh + 2, :] = l[:, cols]
                    sbuf[pc, Dh + 2:ROWS, :] = jnp.zeros(
                        (ROWS - Dh - 2, HLF), jnp.float32)

        def flash_combine(A, R):
            a_m = A[:, Dh:Dh + 1, :]
            r_m = R[:, Dh:Dh + 1, :]
            m_new = jnp.maximum(a_m, r_m)
            aa = jnp.exp(a_m - m_new)
            ar = jnp.exp(r_m - m_new)
            o_new = A[:, 0:Dh, :] * aa + R[:, 0:Dh, :] * ar
            l_new = A[:, Dh + 1:Dh + 2, :] * aa + R[:, Dh + 1:Dh + 2, :] * ar
            return o_new, m_new, l_new

        lo = jnp.int32(0)
        for k in range(LOG2):
            L2 = 16 >> k
            bit = (my >> k) & 1
            klo = lo + bit * L2
            slo = lo + (1 - bit) * L2
            rdma = pltpu.make_async_remote_copy(
                src_ref=sbuf.at[pl.ds(slo, L2)],
                dst_ref=rbuf.at[k, pl.ds(0, L2)],
                send_sem=send_sems.at[k],
                recv_sem=recv_sems.at[k],
                device_id=(my ^ (1 << k),),
                device_id_type=pl.DeviceIdType.MESH,
            )
            rdma.start()
            rdma.wait()
            A = sbuf[pl.ds(klo, L2)]
            R = rbuf[k, pl.ds(0, L2)]
            o_new, m_new, l_new = flash_combine(A, R)
            sbuf[pl.ds(klo, L2), 0:Dh, :] = o_new
            sbuf[pl.ds(klo, L2), Dh:Dh + 1, :] = m_new
            sbuf[pl.ds(klo, L2), Dh + 1:Dh + 2, :] = l_new
            lo = klo

        for k in range(LOG2 - 1, -1, -1):
            pre = 16 >> k
            rdma = pltpu.make_async_remote_copy(
                src_ref=sbuf.at[pl.ds(lo, pre)],
                dst_ref=sbuf.at[pl.ds(lo, pre)],
                send_sem=send_sems.at[LOG2 + (LOG2 - 1 - k)],
                recv_sem=recv_sems.at[LOG2 + (LOG2 - 1 - k)],
                device_id=(my ^ (1 << k),),
                device_id_type=pl.DeviceIdType.MESH,
            )
            rdma.start()
            rdma.wait()
            lo = jnp.minimum(lo, lo ^ pre)

        for b in range(B):
            acc = jnp.zeros((Sq, Dm), dtype=jnp.float32)
            for h in range(Hq):
                i = b * Hq + h
                o = jnp.concatenate(
                    [sbuf[2 * i, 0:Dh, :], sbuf[2 * i + 1, 0:Dh, :]], axis=1)
                l = jnp.concatenate(
                    [sbuf[2 * i, Dh + 1:Dh + 2, :],
                     sbuf[2 * i + 1, Dh + 1:Dh + 2, :]], axis=1)
                acc = acc + lax.dot_general(
                    o / l, wo_ref[h * Dh:(h + 1) * Dh, :],
                    (((0,), (0,)), ((), ())),
                    preferred_element_type=jnp.float32)
            out_ref[b, :, :] = acc

    return pl.pallas_call(
        body,
        out_shape=jax.ShapeDtypeStruct((B, Sq, Dm), jnp.float32),
        in_specs=[pl.BlockSpec(memory_space=pltpu.VMEM)] * 5,
        out_specs=pl.BlockSpec(memory_space=pltpu.VMEM),
        scratch_shapes=[
            pltpu.VMEM((2 * BH, ROWS, HLF), jnp.float32),
            pltpu.VMEM((LOG2, BH, ROWS, HLF), jnp.float32),
            pltpu.SemaphoreType.DMA((2 * LOG2,)),
            pltpu.SemaphoreType.DMA((2 * LOG2,)),
        ],
        compiler_params=pltpu.CompilerParams(collective_id=0),
    )(x, Wq, Wo, K_ext, V_ext)


# device time: 59170 ns/iter; 2.0620x vs baseline; 1.1303x over previous
import jax
import jax.numpy as jnp
from jax import lax
from jax.experimental import pallas as pl
from jax.experimental.pallas import tpu as pltpu

N_DEV = 32
B, Sq, Hq, Hkv, Dh = 2, 256, 8, 2, 64
Dm, Dq = 768, 512
BH = B * Hq
ROWS = 72
HLF = Sq // 2
GQA = Hq // Hkv


def kernel(x, Wq, Wo, K_ext, V_ext):
    def body(x_ref, wq_ref, wo_ref, k_ref, v_ref, out_ref,
             sbuf, rbuf, rs_send, rs_recv, ag_send, ag_recv):
        my = lax.axis_index("i")

        barrier = pltpu.get_barrier_semaphore()
        for t in range(1, N_DEV):
            pl.semaphore_signal(
                barrier, inc=1,
                device_id=(my ^ t,),
                device_id_type=pl.DeviceIdType.MESH,
            )
        pl.semaphore_wait(barrier, N_DEV - 1)

        for b in range(B):
            qb = jnp.dot(x_ref[b], wq_ref[...],
                         preferred_element_type=jnp.float32)
            for h in range(Hq):
                kh = h // GQA
                q = qb[:, h * Dh:(h + 1) * Dh]
                kk = k_ref[b, :, kh, :]
                vv = v_ref[b, :, kh, :]
                sT = lax.dot_general(
                    kk, q, (((1,), (1,)), ((), ())),
                    preferred_element_type=jnp.float32) * 0.125
                m = jnp.max(sT, axis=0, keepdims=True)
                p = jnp.exp(sT - m)
                l = jnp.sum(p, axis=0, keepdims=True)
                oT = lax.dot_general(
                    vv, p, (((0,), (0,)), ((), ())),
                    preferred_element_type=jnp.float32)
                i = b * Hq + h
                for j in range(2):
                    pc = 2 * i + j
                    cols = slice(j * HLF, (j + 1) * HLF)
                    sbuf[pc, 0:Dh, :] = oT[:, cols]
                    sbuf[pc, Dh:Dh + 1, :] = m[:, cols]
                    sbuf[pc, Dh + 1:Dh + 2, :] = l[:, cols]

        def flash_combine(A, R):
            a_m = A[:, Dh:Dh + 1, :]
            r_m = R[:, Dh:Dh + 1, :]
            m_new = jnp.maximum(a_m, r_m)
            aa = jnp.exp(a_m - m_new)
            ar = jnp.exp(r_m - m_new)
            o_new = A[:, 0:Dh, :] * aa + R[:, 0:Dh, :] * ar
            l_new = A[:, Dh + 1:Dh + 2, :] * aa + R[:, Dh + 1:Dh + 2, :] * ar
            return o_new, m_new, l_new

        rs_descs = []
        for t in range(1, N_DEV):
            rdma = pltpu.make_async_remote_copy(
                src_ref=sbuf.at[pl.ds(my ^ t, 1)],
                dst_ref=rbuf.at[pl.ds(t - 1, 1)],
                send_sem=rs_send.at[t - 1],
                recv_sem=rs_recv.at[t - 1],
                device_id=(my ^ t,),
                device_id_type=pl.DeviceIdType.MESH,
            )
            rdma.start()
            rs_descs.append(rdma)
        for rdma in rs_descs:
            rdma.wait_recv()

        rbuf[pl.ds(N_DEV - 1, 1)] = sbuf[pl.ds(my, 1)]
        for L in (16, 8, 4, 2, 1):
            o_new, m_new, l_new = flash_combine(
                rbuf[pl.ds(0, L)], rbuf[pl.ds(L, L)])
            rbuf[pl.ds(0, L), 0:Dh, :] = o_new
            rbuf[pl.ds(0, L), Dh:Dh + 1, :] = m_new
            rbuf[pl.ds(0, L), Dh + 1:Dh + 2, :] = l_new
        sbuf[pl.ds(my, 1), 0:Dh + 2, :] = rbuf[pl.ds(0, 1), 0:Dh + 2, :]

        ag_descs = []
        for t in range(1, N_DEV):
            rdma = pltpu.make_async_remote_copy(
                src_ref=sbuf.at[pl.ds(my, 1)],
                dst_ref=sbuf.at[pl.ds(my, 1)],
                send_sem=ag_send.at[t - 1],
                recv_sem=ag_recv.at[t - 1],
                device_id=(my ^ t,),
                device_id_type=pl.DeviceIdType.MESH,
            )
            rdma.start()
            ag_descs.append(rdma)
        for rdma in ag_descs:
            rdma.wait_recv()
        for rdma in rs_descs + ag_descs:
            rdma.wait_send()

        for b in range(B):
            acc = jnp.zeros((Sq, Dm), dtype=jnp.float32)
            for h in range(Hq):
                i = b * Hq + h
                o = jnp.concatenate(
                    [sbuf[2 * i, 0:Dh, :], sbuf[2 * i + 1, 0:Dh, :]], axis=1)
                l = jnp.concatenate(
                    [sbuf[2 * i, Dh + 1:Dh + 2, :],
                     sbuf[2 * i + 1, Dh + 1:Dh + 2, :]], axis=1)
                acc = acc + lax.dot_general(
                    o / l, wo_ref[h * Dh:(h + 1) * Dh, :],
                    (((0,), (0,)), ((), ())),
                    preferred_element_type=jnp.float32)
            out_ref[b, :, :] = acc

    return pl.pallas_call(
        body,
        out_shape=jax.ShapeDtypeStruct((B, Sq, Dm), jnp.float32),
        in_specs=[pl.BlockSpec(memory_space=pltpu.VMEM)] * 5,
        out_specs=pl.BlockSpec(memory_space=pltpu.VMEM),
        scratch_shapes=[
            pltpu.VMEM((2 * BH, ROWS, HLF), jnp.float32),
            pltpu.VMEM((N_DEV, ROWS, HLF), jnp.float32),
            pltpu.SemaphoreType.DMA((N_DEV - 1,)),
            pltpu.SemaphoreType.DMA((N_DEV - 1,)),
            pltpu.SemaphoreType.DMA((N_DEV - 1,)),
            pltpu.SemaphoreType.DMA((N_DEV - 1,)),
        ],
        compiler_params=pltpu.CompilerParams(collective_id=0),
    )(x, Wq, Wo, K_ext, V_ext)


# device time: 18367 ns/iter; 6.6428x vs baseline; 3.2215x over previous
import os

import jax
import jax.numpy as jnp
from jax import lax
from jax.experimental import pallas as pl
from jax.experimental.pallas import tpu as pltpu

_ABLATE_NOCOMM = os.environ.get("KERNEL_ABLATE", "") == "nocomm"

N_DEV = 32
B, Sq, Hq, Hkv, Dh = 2, 256, 8, 2, 64
Dm, Dq = 768, 512
BH = B * Hq
ROWS = 72
HLF = Sq // 2
GQA = Hq // Hkv


def kernel(x, Wq, Wo, K_ext, V_ext):
    def body(x_ref, wq_ref, wo_ref, k_ref, v_ref, out_ref,
             sbuf, rbuf, rs_send, rs_recv, ag_send, ag_recv):
        my = lax.axis_index("i")

        if not _ABLATE_NOCOMM:
            barrier = pltpu.get_barrier_semaphore()
            for t in range(1, N_DEV):
                pl.semaphore_signal(
                    barrier, inc=1,
                    device_id=(my ^ t,),
                    device_id_type=pl.DeviceIdType.MESH,
                )
            pl.semaphore_wait(barrier, N_DEV - 1)

        for b in range(B):
            qb = jnp.dot(x_ref[b], wq_ref[...],
                         preferred_element_type=jnp.float32)
            for h in range(Hq):
                kh = h // GQA
                q = qb[:, h * Dh:(h + 1) * Dh]
                kk = k_ref[b, :, kh, :]
                vv = v_ref[b, :, kh, :]
                sT = lax.dot_general(
                    kk, q, (((1,), (1,)), ((), ())),
                    preferred_element_type=jnp.float32) * 0.125
                m = jnp.max(sT, axis=0, keepdims=True)
                p = jnp.exp(sT - m)
                l = jnp.sum(p, axis=0, keepdims=True)
                oT = lax.dot_general(
                    vv, p, (((0,), (0,)), ((), ())),
                    preferred_element_type=jnp.float32)
                i = b * Hq + h
                for j in range(2):
                    pc = 2 * i + j
                    cols = slice(j * HLF, (j + 1) * HLF)
                    sbuf[pc, 0:Dh, :] = oT[:, cols]
                    sbuf[pc, Dh:Dh + 1, :] = m[:, cols]
                    sbuf[pc, Dh + 1:Dh + 2, :] = l[:, cols]

        def flash_combine(A, R):
            a_m = A[:, Dh:Dh + 1, :]
            r_m = R[:, Dh:Dh + 1, :]
            m_new = jnp.maximum(a_m, r_m)
            aa = jnp.exp(a_m - m_new)
            ar = jnp.exp(r_m - m_new)
            o_new = A[:, 0:Dh, :] * aa + R[:, 0:Dh, :] * ar
            l_new = A[:, Dh + 1:Dh + 2, :] * aa + R[:, Dh + 1:Dh + 2, :] * ar
            return o_new, m_new, l_new

        rs_descs = []
        if not _ABLATE_NOCOMM:
            for t in range(1, N_DEV):
                rdma = pltpu.make_async_remote_copy(
                    src_ref=sbuf.at[pl.ds(my ^ t, 1)],
                    dst_ref=rbuf.at[pl.ds(t - 1, 1)],
                    send_sem=rs_send.at[t - 1],
                    recv_sem=rs_recv.at[t - 1],
                    device_id=(my ^ t,),
                    device_id_type=pl.DeviceIdType.MESH,
                )
                rdma.start()
                rs_descs.append(rdma)
            for rdma in rs_descs:
                rdma.wait_recv()

        rbuf[pl.ds(N_DEV - 1, 1)] = sbuf[pl.ds(my, 1)]
        for L in (16, 8, 4, 2, 1):
            o_new, m_new, l_new = flash_combine(
                rbuf[pl.ds(0, L)], rbuf[pl.ds(L, L)])
            rbuf[pl.ds(0, L), 0:Dh, :] = o_new
            rbuf[pl.ds(0, L), Dh:Dh + 1, :] = m_new
            rbuf[pl.ds(0, L), Dh + 1:Dh + 2, :] = l_new
        sbuf[pl.ds(my, 1), 0:Dh + 2, :] = rbuf[pl.ds(0, 1), 0:Dh + 2, :]

        ag_descs = []
        if not _ABLATE_NOCOMM:
            for t in range(1, N_DEV):
                rdma = pltpu.make_async_remote_copy(
                    src_ref=sbuf.at[pl.ds(my, 1)],
                    dst_ref=sbuf.at[pl.ds(my, 1)],
                    send_sem=ag_send.at[t - 1],
                    recv_sem=ag_recv.at[t - 1],
                    device_id=(my ^ t,),
                    device_id_type=pl.DeviceIdType.MESH,
                )
                rdma.start()
                ag_descs.append(rdma)
            for rdma in ag_descs:
                rdma.wait_recv()
            for rdma in rs_descs + ag_descs:
                rdma.wait_send()

        for b in range(B):
            acc = jnp.zeros((Sq, Dm), dtype=jnp.float32)
            for h in range(Hq):
                i = b * Hq + h
                o = jnp.concatenate(
                    [sbuf[2 * i, 0:Dh, :], sbuf[2 * i + 1, 0:Dh, :]], axis=1)
                l = jnp.concatenate(
                    [sbuf[2 * i, Dh + 1:Dh + 2, :],
                     sbuf[2 * i + 1, Dh + 1:Dh + 2, :]], axis=1)
                acc = acc + lax.dot_general(
                    o / l, wo_ref[h * Dh:(h + 1) * Dh, :],
                    (((0,), (0,)), ((), ())),
                    preferred_element_type=jnp.float32)
            out_ref[b, :, :] = acc

    return pl.pallas_call(
        body,
        out_shape=jax.ShapeDtypeStruct((B, Sq, Dm), jnp.float32),
        in_specs=[pl.BlockSpec(memory_space=pltpu.VMEM)] * 5,
        out_specs=pl.BlockSpec(memory_space=pltpu.VMEM),
        scratch_shapes=[
            pltpu.VMEM((2 * BH, ROWS, HLF), jnp.float32),
            pltpu.VMEM((N_DEV, ROWS, HLF), jnp.float32),
            pltpu.SemaphoreType.DMA((N_DEV - 1,)),
            pltpu.SemaphoreType.DMA((N_DEV - 1,)),
            pltpu.SemaphoreType.DMA((N_DEV - 1,)),
            pltpu.SemaphoreType.DMA((N_DEV - 1,)),
        ],
        compiler_params=(None if _ABLATE_NOCOMM
                         else pltpu.CompilerParams(collective_id=0)),
    )(x, Wq, Wo, K_ext, V_ext)
